# Initial kernel scaffold; baseline (speedup 1.0000x reference)
import jax
import jax.numpy as jnp
from jax import lax
from jax.experimental import pallas as pl
from jax.experimental.pallas import tpu as pltpu
from jax.experimental.pallas import tpu_sc as plsc

N = 10000
E = 320000
HID = 128
HALF = 64
NUM_LAYER = 3

NCORE = 2
NSUB = 16
CHUNK_ROWS = 4
E_PAD = 327680
ROWS = E_PAD // 128
ROWS_PER_TILE = ROWS // NSUB
CHUNKS_PER_TILE = ROWS_PER_TILE // CHUNK_ROWS
NODE_ROWS_PER_TILE = N // NSUB


def _embed_nodes_body(x_ref, emb_ref, out_ref):
    xi = x_ref[...]
    oh = (xi == lax.broadcasted_iota(jnp.int32, (1, 101), 1)).astype(jnp.float32)
    h = jnp.dot(oh, emb_ref[...], preferred_element_type=jnp.float32,
                precision=lax.Precision.HIGHEST)
    out_ref[0, :, :] = h[:, :HALF]
    out_ref[1, :, :] = h[:, HALF:]


def _embed_nodes(x, emb):
    B = 2000
    return pl.pallas_call(
        _embed_nodes_body,
        grid=(N // B,),
        in_specs=[
            pl.BlockSpec((B, 1), lambda i: (i, 0)),
            pl.BlockSpec((101, HID), lambda i: (0, 0)),
        ],
        out_specs=pl.BlockSpec((2, B, HALF), lambda i: (0, i, 0)),
        out_shape=jax.ShapeDtypeStruct((2, N, HALF), jnp.float32),
    )(x, emb)


def _bond_body(ea_ref, tab_ref, out_ref):
    i = pl.program_id(0)
    ea = ea_ref[...]
    cols = lax.broadcasted_iota(jnp.int32, (1, 51), 1)
    oh = jnp.zeros((ea.shape[0], 51), jnp.float32)
    for t in range(3):
        sel = (cols >= 17 * t) & (cols < 17 * (t + 1)) & (ea[:, t:t + 1] == cols - 17 * t)
        oh = oh + sel.astype(jnp.float32)
    bond = jnp.dot(oh, tab_ref[...], preferred_element_type=jnp.float32,
                   precision=lax.Precision.HIGH)
    row = i * ea.shape[0] + lax.broadcasted_iota(jnp.int32, (ea.shape[0], 1), 0)
    bond = bond - jnp.where(row >= E, jnp.float32(1e30), jnp.float32(0.0))
    out_ref[0, :, :] = bond[:, :HALF]
    out_ref[1, :, :] = bond[:, HALF:]


def _bond(ea_pad, tables):
    B = 2048
    return pl.pallas_call(
        _bond_body,
        grid=(E_PAD // B,),
        in_specs=[
            pl.BlockSpec((B, 3), lambda i: (i, 0)),
            pl.BlockSpec((51, HID), lambda i: (0, 0)),
        ],
        out_specs=pl.BlockSpec((2, B, HALF), lambda i: (0, i, 0)),
        out_shape=jax.ShapeDtypeStruct((2, E_PAD, HALF), jnp.float32),
    )(ea_pad, tables)


def _dense_body(z_ref, h_ref, w1_ref, b1_ref, w2_ref, b2_ref, g_ref, bb_ref,
                out_ref):
    z = jnp.concatenate([z_ref[0, :, :], z_ref[1, :, :]], axis=-1)
    t = jnp.dot(z, w1_ref[...], preferred_element_type=jnp.float32,
                precision=lax.Precision.HIGHEST) + b1_ref[...]
    t = jnp.maximum(t, 0.0)
    t = jnp.dot(t, w2_ref[...], preferred_element_type=jnp.float32,
                precision=lax.Precision.HIGHEST) + b2_ref[...]
    mu = jnp.mean(t, axis=-1, keepdims=True)
    var = jnp.mean((t - mu) ** 2, axis=-1, keepdims=True)
    t = (t - mu) * lax.rsqrt(var + 1e-5) * g_ref[...] + bb_ref[...]
    t = jnp.maximum(t, 0.0)
    h = jnp.concatenate([h_ref[0, :, :], h_ref[1, :, :]], axis=-1)
    o = h + t
    out_ref[0, :, :] = o[:, :HALF]
    out_ref[1, :, :] = o[:, HALF:]


def _dense(z2, h2, w1, b1, w2, b2, g, bb):
    B = 2000
    vec = lambda: pl.BlockSpec((1, HID), lambda i: (0, 0))
    return pl.pallas_call(
        _dense_body,
        grid=(N // B,),
        in_specs=[
            pl.BlockSpec((2, B, HALF), lambda i: (0, i, 0)),
            pl.BlockSpec((2, B, HALF), lambda i: (0, i, 0)),
            pl.BlockSpec((HID, HID), lambda i: (0, 0)), vec(),
            pl.BlockSpec((HID, HID), lambda i: (0, 0)), vec(),
            vec(), vec(),
        ],
        out_specs=pl.BlockSpec((2, B, HALF), lambda i: (0, i, 0)),
        out_shape=jax.ShapeDtypeStruct((2, N, HALF), jnp.float32),
    )(z2, h2, w1, b1, w2, b2, g, bb)


def _edge_body(h_hbm, bond_hbm, src_hbm, dst_hbm, z_hbm,
               shared_z, src_v, dst_v, bond_v, hrow_v, sem):
    c = lax.axis_index("c")
    s = lax.axis_index("s")

    nbase = s * NODE_ROWS_PER_TILE
    flat = hrow_v.reshape(CHUNK_ROWS * 128, HALF)
    for off, cnt in ((0, 512), (512, NODE_ROWS_PER_TILE - 512)):
        pltpu.sync_copy(h_hbm.at[pl.ds(c * N + nbase + off, cnt)],
                        flat.at[pl.ds(0, cnt)])
        pltpu.sync_copy(flat.at[pl.ds(0, cnt)],
                        shared_z.at[pl.ds(nbase + off, cnt)])
    plsc.subcore_barrier()

    @pl.loop(0, CHUNKS_PER_TILE)
    def _chunk(i):
        row = s * ROWS_PER_TILE + i * CHUNK_ROWS
        pltpu.sync_copy(src_hbm.at[c, pl.ds(row, CHUNK_ROWS)], src_v)
        pltpu.sync_copy(dst_hbm.at[pl.ds(row, CHUNK_ROWS)], dst_v)
        pltpu.sync_copy(bond_hbm.at[c, pl.ds(row, CHUNK_ROWS)], bond_v)
        for j in range(CHUNK_ROWS):
            pltpu.async_copy(h_hbm.at[src_v.at[j]], hrow_v.at[j], sem).wait()

        @pl.loop(0, 128)
        def _row(k):
            for j in range(CHUNK_ROWS):
                for cc in range(HALF // 16):
                    sl = pl.ds(cc * 16, 16)
                    bond_v[j, k, sl] = jnp.maximum(
                        bond_v[j, k, sl] + hrow_v[j, k, sl], 0.0)

        for j in range(CHUNK_ROWS):
            pltpu.sync_copy(bond_v.at[j], shared_z.at[dst_v.at[j]], add=True)

    plsc.subcore_barrier()
    for off, cnt in ((0, 512), (512, NODE_ROWS_PER_TILE - 512)):
        pltpu.sync_copy(shared_z.at[pl.ds(nbase + off, cnt)],
                        flat.at[pl.ds(0, cnt)])
        pltpu.sync_copy(flat.at[pl.ds(0, cnt)],
                        z_hbm.at[pl.ds(c * N + nbase + off, cnt)])


def _edge_pass(h_flat, bond3, srcb, dstb):
    mesh = plsc.VectorSubcoreMesh(core_axis_name="c", subcore_axis_name="s")
    f = pl.kernel(
        _edge_body,
        out_type=jax.ShapeDtypeStruct((2 * N, HALF), jnp.float32),
        mesh=mesh,
        scratch_types=[
            pltpu.VMEM_SHARED((N, HALF), jnp.float32),
            pltpu.VMEM((CHUNK_ROWS, 128), jnp.int32),
            pltpu.VMEM((CHUNK_ROWS, 128), jnp.int32),
            pltpu.VMEM((CHUNK_ROWS, 128, HALF), jnp.float32),
            pltpu.VMEM((CHUNK_ROWS, 128, HALF), jnp.float32),
            pltpu.SemaphoreType.DMA,
        ],
    )
    return f(h_flat, bond3, srcb, dstb)


def kernel(x, edge_index, edge_attr, subgs, num_subg, num_node, num_edge,
           inmlp_emb, edge_emb1, edge_emb2, edge_emb3,
           W1, b1, W2, b2, ln_g, ln_b):
    tables = jnp.concatenate([edge_emb1, edge_emb2, edge_emb3], axis=0)

    src = jnp.pad(edge_index[0], (0, E_PAD - E))
    dst = jnp.pad(edge_index[1], (0, E_PAD - E))
    ea_pad = jnp.pad(edge_attr, ((0, E_PAD - E), (0, 0)))

    srcb = jnp.stack([src, src + N]).reshape(2, ROWS, 128)
    dstb = dst.reshape(ROWS, 128)

    h2 = _embed_nodes(x, inmlp_emb)
    bond3 = _bond(ea_pad, tables).reshape(2, ROWS, 128, HALF)

    for l in range(NUM_LAYER):
        h_flat = h2.reshape(2 * N, HALF)
        z = _edge_pass(h_flat, bond3, srcb, dstb)
        h2 = _dense(z.reshape(2, N, HALF), h2,
                    W1[l], b1[l].reshape(1, HID), W2[l], b2[l].reshape(1, HID),
                    ln_g[l].reshape(1, HID), ln_b[l].reshape(1, HID))

    return jnp.concatenate([h2[0], h2[1]], axis=-1)


# speedup vs baseline: 2.4203x; 2.4203x over previous
import jax
import jax.numpy as jnp
from jax import lax
from jax.experimental import pallas as pl
from jax.experimental.pallas import tpu as pltpu
from jax.experimental.pallas import tpu_sc as plsc

N = 10000
E = 320000
HID = 128
NUM_LAYER = 3

NCORE = 2
NSUB = 16
CHUNK_ROWS = 1
E_PAD = 327680
ROWS = E_PAD // 128
ROWS_PER_CORE = ROWS // NCORE
ROWS_PER_TILE = ROWS_PER_CORE // NSUB
CHUNKS_PER_TILE = ROWS_PER_TILE // CHUNK_ROWS
CE = CHUNK_ROWS * 128


def _embed_nodes_body(x_ref, emb_ref, out_ref):
    xi = x_ref[...]
    oh = (xi == lax.broadcasted_iota(jnp.int32, (1, 101), 1)).astype(jnp.float32)
    out_ref[...] = jnp.dot(oh, emb_ref[...], preferred_element_type=jnp.float32,
                           precision=lax.Precision.HIGHEST)


def _embed_nodes(x, emb):
    B = 2000
    return pl.pallas_call(
        _embed_nodes_body,
        grid=(N // B,),
        in_specs=[
            pl.BlockSpec((B, 1), lambda i: (i, 0)),
            pl.BlockSpec((101, HID), lambda i: (0, 0)),
        ],
        out_specs=pl.BlockSpec((B, HID), lambda i: (i, 0)),
        out_shape=jax.ShapeDtypeStruct((N, HID), jnp.float32),
    )(x, emb)


def _bond_body(ea_ref, tab_ref, out_ref):
    i = pl.program_id(0)
    ea = ea_ref[...]
    cols = lax.broadcasted_iota(jnp.int32, (1, 51), 1)
    oh = jnp.zeros((ea.shape[0], 51), jnp.float32)
    for t in range(3):
        sel = (cols >= 17 * t) & (cols < 17 * (t + 1)) & (ea[:, t:t + 1] == cols - 17 * t)
        oh = oh + sel.astype(jnp.float32)
    bond = jnp.dot(oh, tab_ref[...], preferred_element_type=jnp.float32,
                   precision=lax.Precision.HIGHEST)
    row = i * ea.shape[0] + lax.broadcasted_iota(jnp.int32, (ea.shape[0], 1), 0)
    out_ref[...] = bond - jnp.where(row >= E, jnp.float32(1e30), jnp.float32(0.0))


def _bond(ea_pad, tables):
    B = 2048
    return pl.pallas_call(
        _bond_body,
        grid=(E_PAD // B,),
        in_specs=[
            pl.BlockSpec((B, 3), lambda i: (i, 0)),
            pl.BlockSpec((51, HID), lambda i: (0, 0)),
        ],
        out_specs=pl.BlockSpec((B, HID), lambda i: (i, 0)),
        out_shape=jax.ShapeDtypeStruct((E_PAD, HID), jnp.float32),
    )(ea_pad, tables)


def _dense_body(a_ref, h_ref, w1_ref, b1_ref, w2_ref, b2_ref, g_ref, bb_ref,
                out_ref):
    h = h_ref[...]
    z = a_ref[0, :, :] + a_ref[1, :, :] - h
    t = jnp.dot(z, w1_ref[...], preferred_element_type=jnp.float32,
                precision=lax.Precision.HIGHEST) + b1_ref[...]
    t = jnp.maximum(t, 0.0)
    t = jnp.dot(t, w2_ref[...], preferred_element_type=jnp.float32,
                precision=lax.Precision.HIGHEST) + b2_ref[...]
    mu = jnp.mean(t, axis=-1, keepdims=True)
    var = jnp.mean((t - mu) ** 2, axis=-1, keepdims=True)
    t = (t - mu) * lax.rsqrt(var + 1e-5) * g_ref[...] + bb_ref[...]
    t = jnp.maximum(t, 0.0)
    out_ref[...] = h + t


def _dense(a2, h, w1, b1, w2, b2, g, bb):
    B = 2000
    vec = lambda: pl.BlockSpec((1, HID), lambda i: (0, 0))
    return pl.pallas_call(
        _dense_body,
        grid=(N // B,),
        in_specs=[
            pl.BlockSpec((2, B, HID), lambda i: (0, i, 0)),
            pl.BlockSpec((B, HID), lambda i: (i, 0)),
            pl.BlockSpec((HID, HID), lambda i: (0, 0)), vec(),
            pl.BlockSpec((HID, HID), lambda i: (0, 0)), vec(),
            vec(), vec(),
        ],
        out_specs=pl.BlockSpec((B, HID), lambda i: (i, 0)),
        out_shape=jax.ShapeDtypeStruct((N, HID), jnp.float32),
    )(a2, h, w1, b1, w2, b2, g, bb)


def _edge_body(h_hbm, bond_hbm, src_hbm, dst_hbm, a_hbm,
               shared_z, src_v, dst_v, bond_v, hrow_v, sem):
    c = lax.axis_index("c")
    s = lax.axis_index("s")

    @pl.when(s == 0)
    def _stage():
        pltpu.sync_copy(h_hbm, shared_z)

    plsc.subcore_barrier()

    @pl.loop(0, CHUNKS_PER_TILE)
    def _chunk(i):
        row = c * ROWS_PER_CORE + s * ROWS_PER_TILE + i * CHUNK_ROWS
        ebase = pl.multiple_of(row * 128, 8)
        pltpu.sync_copy(src_hbm.at[pl.ds(row, CHUNK_ROWS)], src_v)
        pltpu.sync_copy(dst_hbm.at[pl.ds(row, CHUNK_ROWS)], dst_v)
        pltpu.sync_copy(bond_hbm.at[pl.ds(ebase, CE)], bond_v)
        for j in range(CHUNK_ROWS):
            pltpu.async_copy(h_hbm.at[src_v.at[j]],
                             hrow_v.at[pl.ds(j * 128, 128)], sem).wait()

        @pl.loop(0, CE, step=2)
        def _row(k):
            for j in range(2):
                for cc in range(HID // 16):
                    sl = pl.ds(cc * 16, 16)
                    bond_v[k + j, sl] = jnp.maximum(
                        bond_v[k + j, sl] + hrow_v[k + j, sl], 0.0)

        for j in range(CHUNK_ROWS):
            pltpu.sync_copy(bond_v.at[pl.ds(j * 128, 128)],
                            shared_z.at[dst_v.at[j]], add=True)

    plsc.subcore_barrier()

    @pl.when(s == 0)
    def _readout():
        pltpu.sync_copy(shared_z, a_hbm.at[c])


def _edge_pass(h, bond, srcb, dstb):
    mesh = plsc.VectorSubcoreMesh(core_axis_name="c", subcore_axis_name="s",
                                  num_cores=NCORE, num_subcores=NSUB)
    f = pl.kernel(
        _edge_body,
        out_type=jax.ShapeDtypeStruct((2, N, HID), jnp.float32),
        mesh=mesh,
        scratch_types=[
            pltpu.VMEM_SHARED((N, HID), jnp.float32),
            pltpu.VMEM((CHUNK_ROWS, 128), jnp.int32),
            pltpu.VMEM((CHUNK_ROWS, 128), jnp.int32),
            pltpu.VMEM((CE, HID), jnp.float32),
            pltpu.VMEM((CE, HID), jnp.float32),
            pltpu.SemaphoreType.DMA,
        ],
    )
    return f(h, bond, srcb, dstb)


def kernel(x, edge_index, edge_attr, subgs, num_subg, num_node, num_edge,
           inmlp_emb, edge_emb1, edge_emb2, edge_emb3,
           W1, b1, W2, b2, ln_g, ln_b):
    tables = jnp.concatenate([edge_emb1, edge_emb2, edge_emb3], axis=0)

    srcb = jnp.pad(edge_index[0], (0, E_PAD - E)).reshape(ROWS, 128)
    dstb = jnp.pad(edge_index[1], (0, E_PAD - E)).reshape(ROWS, 128)
    ea_pad = jnp.pad(edge_attr, ((0, E_PAD - E), (0, 0)))

    h = _embed_nodes(x, inmlp_emb)
    bond = _bond(ea_pad, tables)

    for l in range(NUM_LAYER):
        a2 = _edge_pass(h, bond, srcb, dstb)
        h = _dense(a2, h,
                   W1[l], b1[l].reshape(1, HID), W2[l], b2[l].reshape(1, HID),
                   ln_g[l].reshape(1, HID), ln_b[l].reshape(1, HID))

    return h


# speedup vs baseline: 2.8579x; 1.1808x over previous
import jax
import jax.numpy as jnp
from jax import lax
from jax.experimental import pallas as pl
from jax.experimental.pallas import tpu as pltpu
from jax.experimental.pallas import tpu_sc as plsc

N = 10000
E = 320000
HID = 128
NUM_LAYER = 3

NCORE = 2
NSUB = 16
CHUNK_ROWS = 1
E_PAD = 327680
ROWS = E_PAD // 128
ROWS_PER_CORE = ROWS // NCORE
ROWS_PER_TILE = ROWS_PER_CORE // NSUB
CHUNKS_PER_TILE = ROWS_PER_TILE // CHUNK_ROWS
CE = CHUNK_ROWS * 128


def _embed_nodes_body(x_ref, emb_ref, out_ref):
    xi = x_ref[...]
    oh = (xi == lax.broadcasted_iota(jnp.int32, (1, 101), 1)).astype(jnp.float32)
    out_ref[...] = jnp.dot(oh, emb_ref[...], preferred_element_type=jnp.float32,
                           precision=lax.Precision.HIGHEST)


def _embed_nodes(x, emb):
    B = 2000
    return pl.pallas_call(
        _embed_nodes_body,
        grid=(N // B,),
        in_specs=[
            pl.BlockSpec((B, 1), lambda i: (i, 0)),
            pl.BlockSpec((101, HID), lambda i: (0, 0)),
        ],
        out_specs=pl.BlockSpec((B, HID), lambda i: (i, 0)),
        out_shape=jax.ShapeDtypeStruct((N, HID), jnp.float32),
    )(x, emb)


def _bond_body(ea_ref, tab_ref, out_ref):
    i = pl.program_id(0)
    ea = ea_ref[...]
    cols = lax.broadcasted_iota(jnp.int32, (1, 51), 1)
    oh = jnp.zeros((ea.shape[0], 51), jnp.float32)
    for t in range(3):
        sel = (cols >= 17 * t) & (cols < 17 * (t + 1)) & (ea[:, t:t + 1] == cols - 17 * t)
        oh = oh + sel.astype(jnp.float32)
    bond = jnp.dot(oh, tab_ref[...], preferred_element_type=jnp.float32,
                   precision=lax.Precision.HIGHEST)
    row = i * ea.shape[0] + lax.broadcasted_iota(jnp.int32, (ea.shape[0], 1), 0)
    out_ref[...] = bond - jnp.where(row >= E, jnp.float32(1e30), jnp.float32(0.0))


def _bond(ea_pad, tables):
    B = 2048
    return pl.pallas_call(
        _bond_body,
        grid=(E_PAD // B,),
        in_specs=[
            pl.BlockSpec((B, 3), lambda i: (i, 0)),
            pl.BlockSpec((51, HID), lambda i: (0, 0)),
        ],
        out_specs=pl.BlockSpec((B, HID), lambda i: (i, 0)),
        out_shape=jax.ShapeDtypeStruct((E_PAD, HID), jnp.float32),
    )(ea_pad, tables)


def _dense_body(a_ref, h_ref, w1_ref, b1_ref, w2_ref, b2_ref, g_ref, bb_ref,
                out_ref):
    h = h_ref[...]
    z = a_ref[0, :, :] + a_ref[1, :, :] - h
    t = jnp.dot(z, w1_ref[...], preferred_element_type=jnp.float32,
                precision=lax.Precision.HIGHEST) + b1_ref[...]
    t = jnp.maximum(t, 0.0)
    t = jnp.dot(t, w2_ref[...], preferred_element_type=jnp.float32,
                precision=lax.Precision.HIGHEST) + b2_ref[...]
    mu = jnp.mean(t, axis=-1, keepdims=True)
    var = jnp.mean((t - mu) ** 2, axis=-1, keepdims=True)
    t = (t - mu) * lax.rsqrt(var + 1e-5) * g_ref[...] + bb_ref[...]
    t = jnp.maximum(t, 0.0)
    out_ref[...] = h + t


def _dense(a2, h, w1, b1, w2, b2, g, bb):
    B = 2000
    vec = lambda: pl.BlockSpec((1, HID), lambda i: (0, 0))
    return pl.pallas_call(
        _dense_body,
        grid=(N // B,),
        in_specs=[
            pl.BlockSpec((2, B, HID), lambda i: (0, i, 0)),
            pl.BlockSpec((B, HID), lambda i: (i, 0)),
            pl.BlockSpec((HID, HID), lambda i: (0, 0)), vec(),
            pl.BlockSpec((HID, HID), lambda i: (0, 0)), vec(),
            vec(), vec(),
        ],
        out_specs=pl.BlockSpec((B, HID), lambda i: (i, 0)),
        out_shape=jax.ShapeDtypeStruct((N, HID), jnp.float32),
    )(a2, h, w1, b1, w2, b2, g, bb)


def _edge_body(h_hbm, bond_hbm, sd_hbm, a_hbm,
               shared_z, idx0, idx1, hrow0, hrow1, bond_v,
               sg0, sg1, sb, ss0, ss1, si):
    c = lax.axis_index("c")
    s = lax.axis_index("s")
    idxs = (idx0, idx1)
    hrows = (hrow0, hrow1)
    sgs = (sg0, sg1)
    sss = (ss0, ss1)
    row0 = c * ROWS_PER_CORE + s * ROWS_PER_TILE

    @pl.when(s == 0)
    def _stage():
        pltpu.sync_copy(h_hbm, shared_z)

    plsc.subcore_barrier()

    def bond_slice(r):
        return bond_hbm.at[pl.ds(pl.multiple_of(r * 128, 8), CE)]

    pltpu.sync_copy(sd_hbm.at[row0], idx0)
    pltpu.async_copy(h_hbm.at[idx0.at[0]], hrow0, sg0)
    pltpu.async_copy(bond_slice(row0), bond_v, sb)

    def chunk(ci, b):
        hb, hn = hrows[b], hrows[1 - b]
        ib, inx = idxs[b], idxs[1 - b]
        pltpu.make_async_copy(h_hbm.at[ib.at[0]], hb, sgs[b]).wait()
        pltpu.make_async_copy(bond_slice(row0), bond_v, sb).wait()

        def wait_prev_scatter():
            pltpu.make_async_copy(hn, shared_z.at[inx.at[1]],
                                  sss[1 - b]).wait()
        if b == 1:
            wait_prev_scatter()
        else:
            pl.when(ci > 0)(wait_prev_scatter)

        nrow = row0 + jnp.minimum(ci + 1, CHUNKS_PER_TILE - 1)
        pltpu.async_copy(sd_hbm.at[nrow], inx, si)

        @pl.loop(0, CE, step=2)
        def _row(k):
            for j in range(2):
                for cc in range(HID // 16):
                    sl = pl.ds(cc * 16, 16)
                    hb[k + j, sl] = jnp.maximum(
                        hb[k + j, sl] + bond_v[k + j, sl], 0.0)

        pltpu.async_copy(bond_slice(nrow), bond_v, sb)
        pltpu.make_async_copy(sd_hbm.at[nrow], inx, si).wait()
        pltpu.async_copy(h_hbm.at[inx.at[0]], hn, sgs[1 - b])
        pltpu.async_copy(hb, shared_z.at[ib.at[1]], sss[b], add=True)

    @pl.loop(0, CHUNKS_PER_TILE, step=2)
    def _pair(i):
        chunk(i, 0)
        chunk(i + 1, 1)

    pltpu.make_async_copy(h_hbm.at[idx0.at[0]], hrow0, sg0).wait()
    pltpu.make_async_copy(bond_slice(row0), bond_v, sb).wait()
    pltpu.make_async_copy(hrow1, shared_z.at[idx1.at[1]], ss1).wait()

    plsc.subcore_barrier()

    @pl.when(s == 0)
    def _readout():
        pltpu.sync_copy(shared_z, a_hbm.at[c])


def _edge_pass(h, bond, sdb):
    mesh = plsc.VectorSubcoreMesh(core_axis_name="c", subcore_axis_name="s",
                                  num_cores=NCORE, num_subcores=NSUB)
    f = pl.kernel(
        _edge_body,
        out_type=jax.ShapeDtypeStruct((2, N, HID), jnp.float32),
        mesh=mesh,
        scratch_types=[
            pltpu.VMEM_SHARED((N, HID), jnp.float32),
            pltpu.VMEM((2, 128), jnp.int32),
            pltpu.VMEM((2, 128), jnp.int32),
            pltpu.VMEM((CE, HID), jnp.float32),
            pltpu.VMEM((CE, HID), jnp.float32),
            pltpu.VMEM((CE, HID), jnp.float32),
            pltpu.SemaphoreType.DMA,
            pltpu.SemaphoreType.DMA,
            pltpu.SemaphoreType.DMA,
            pltpu.SemaphoreType.DMA,
            pltpu.SemaphoreType.DMA,
            pltpu.SemaphoreType.DMA,
        ],
    )
    return f(h, bond, sdb)


def kernel(x, edge_index, edge_attr, subgs, num_subg, num_node, num_edge,
           inmlp_emb, edge_emb1, edge_emb2, edge_emb3,
           W1, b1, W2, b2, ln_g, ln_b):
    tables = jnp.concatenate([edge_emb1, edge_emb2, edge_emb3], axis=0)

    srcb = jnp.pad(edge_index[0], (0, E_PAD - E)).reshape(ROWS, 128)
    dstb = jnp.pad(edge_index[1], (0, E_PAD - E)).reshape(ROWS, 128)
    sdb = jnp.stack([srcb, dstb], axis=1)
    ea_pad = jnp.pad(edge_attr, ((0, E_PAD - E), (0, 0)))

    h = _embed_nodes(x, inmlp_emb)
    bond = _bond(ea_pad, tables)

    for l in range(NUM_LAYER):
        a2 = _edge_pass(h, bond, sdb)
        h = _dense(a2, h,
                   W1[l], b1[l].reshape(1, HID), W2[l], b2[l].reshape(1, HID),
                   ln_g[l].reshape(1, HID), ln_b[l].reshape(1, HID))

    return h
